# baseline (speedup 1.0000x reference)
import functools

import jax
import jax.numpy as jnp
from jax import lax
from jax.experimental import pallas as pl
from jax.experimental.pallas import tpu as pltpu
from jax.experimental.pallas import tpu_sc as plsc

TABLE_N = 256
BETA = 0.2

NC, NS, L = 2, 16, 16
NW = NC * NS

ROWS = 16384
COLS = 2048
N_TOTAL = ROWS * COLS

TC_LOOKUP_ROWS = 3072
SC_ROWS = ROWS - TC_LOOKUP_ROWS
SC_ROWS_PER_W = SC_ROWS // NW
CH_ROWS = 8
SC_NCHUNK = SC_ROWS_PER_W // CH_ROWS
VECS_PER_ROW = COLS // L

TC_BM = 512
TC_GRID = ROWS // TC_BM
TC_LOOKUP_BLOCKS = TC_LOOKUP_ROWS // TC_BM
SUB = 8


def _sc_lookup_body(direct_hbm, table_hbm, out_hbm, table_v, buf0, buf1,
                    out_v, sem0, sem1):
    wid = lax.axis_index("s") * NC + lax.axis_index("c")
    pltpu.sync_copy(table_hbm, table_v)
    row0 = TC_LOOKUP_ROWS + wid * SC_ROWS_PER_W

    def src(k):
        kc = jnp.minimum(k, SC_NCHUNK - 1)
        return direct_hbm.at[pl.ds(row0 + kc * CH_ROWS, CH_ROWS), :]

    def compute(buf, a):
        for r in range(CH_ROWS):
            def vec_body(i, acc):
                d = buf[r, pl.ds(i * L, L)]
                idx = jnp.clip(d * 255.0, 0.0, 255.0).astype(jnp.int32)
                t = plsc.load_gather(table_v, [idx])
                diff = d - t
                return acc + diff * diff

            a = plsc.parallel_loop(0, VECS_PER_ROW, 1, unroll=8,
                                   carry=a)(vec_body)
        return a

    pltpu.async_copy(src(0), buf0, sem0)
    pltpu.async_copy(src(1), buf1, sem1)

    def outer(k2, acc):
        k = 2 * k2
        pltpu.make_async_copy(src(k), buf0, sem0).wait()
        acc = compute(buf0, acc)
        pltpu.async_copy(src(k + 2), buf0, sem0)
        pltpu.make_async_copy(src(k + 1), buf1, sem1).wait()
        acc = compute(buf1, acc)
        pltpu.async_copy(src(k + 3), buf1, sem1)
        return acc

    acc = lax.fori_loop(0, SC_NCHUNK // 2, outer, jnp.zeros((L,), jnp.float32))
    pltpu.make_async_copy(src(SC_NCHUNK), buf0, sem0).wait()
    pltpu.make_async_copy(src(SC_NCHUNK + 1), buf1, sem1).wait()
    out_v[...] = acc
    pltpu.sync_copy(out_v, out_hbm.at[wid])


def _tc_body(d_ref, tbl_ref, out_ref, acc_ref):
    d = d_ref[...]
    r = d * 50.0
    neg = jnp.where(d > -BETA, r * r, -1000.0 * d - 100.0)
    p = jnp.where(d >= 0.0, d, neg)
    s = jnp.sum(p)

    @pl.when(pl.program_id(0) == 0)
    def _init():
        acc_ref[0] = 0.0

    acc_ref[0] += s

    @pl.when(pl.program_id(0) < TC_LOOKUP_BLOCKS)
    def _lookup():
        tlo = tbl_ref[:, :128]
        thi = tbl_ref[:, 128:]

        def row_body(rr, s2):
            d8 = d_ref[rr]
            idx = jnp.clip(d8 * 255.0, 0.0, 255.0).astype(jnp.int32)
            low = idx & 127
            glo = jnp.take_along_axis(tlo, low, axis=1)
            ghi = jnp.take_along_axis(thi, low, axis=1)
            t = jnp.where(idx < 128, glo, ghi)
            diff = d8 - t
            return s2 + jnp.sum(diff * diff)

        acc_ref[0] += lax.fori_loop(0, TC_BM // SUB, row_body,
                                    jnp.float32(0.0))

    @pl.when(pl.program_id(0) == pl.num_programs(0) - 1)
    def _fin():
        out_ref[0] = acc_ref[0]


@jax.jit
def kernel(direct, table):
    d2 = direct.reshape(ROWS, COLS)

    sc_partials = pl.kernel(
        _sc_lookup_body,
        out_type=jax.ShapeDtypeStruct((NW, L), jnp.float32),
        mesh=plsc.VectorSubcoreMesh(
            core_axis_name="c", subcore_axis_name="s",
            num_cores=NC, num_subcores=NS),
        scratch_types=[
            pltpu.VMEM((TABLE_N,), jnp.float32),
            pltpu.VMEM((CH_ROWS, COLS), jnp.float32),
            pltpu.VMEM((CH_ROWS, COLS), jnp.float32),
            pltpu.VMEM((L,), jnp.float32),
            pltpu.SemaphoreType.DMA,
            pltpu.SemaphoreType.DMA,
        ],
        compiler_params=pltpu.CompilerParams(needs_layout_passes=False),
    )(d2, table)

    tbl8 = jnp.broadcast_to(table, (SUB, TABLE_N))
    d3 = direct.reshape(ROWS // SUB, SUB, COLS)
    tc_sum = pl.pallas_call(
        _tc_body,
        grid=(TC_GRID,),
        in_specs=[
            pl.BlockSpec((TC_BM // SUB, SUB, COLS), lambda i: (i, 0, 0)),
            pl.BlockSpec((SUB, TABLE_N), lambda i: (0, 0)),
        ],
        out_specs=pl.BlockSpec(memory_space=pltpu.SMEM),
        out_shape=jax.ShapeDtypeStruct((1,), jnp.float32),
        scratch_shapes=[pltpu.SMEM((1,), jnp.float32)],
    )(d3, tbl8)

    return (jnp.sum(sc_partials) + tc_sum[0]) / N_TOTAL


# speedup vs baseline: 1.0827x; 1.0827x over previous
import functools

import jax
import jax.numpy as jnp
from jax import lax
from jax.experimental import pallas as pl
from jax.experimental.pallas import tpu as pltpu
from jax.experimental.pallas import tpu_sc as plsc

TABLE_N = 256
BETA = 0.2

NC, NS, L = 2, 16, 16
NW = NC * NS

ROWS = 16384
COLS = 2048
N_TOTAL = ROWS * COLS

TC_LOOKUP_ROWS = 2560
SC_ROWS = ROWS - TC_LOOKUP_ROWS
SC_ROWS_PER_W = SC_ROWS // NW
CH_ROWS = 8
SC_NCHUNK = SC_ROWS_PER_W // CH_ROWS
VECS_PER_ROW = COLS // L

TC_BM = 512
TC_GRID = ROWS // TC_BM
TC_LOOKUP_BLOCKS = TC_LOOKUP_ROWS // TC_BM
SUB = 8


def _sc_lookup_body(direct_hbm, table_hbm, out_hbm, table_v, buf0, buf1,
                    out_v, sem0, sem1):
    wid = lax.axis_index("s") * NC + lax.axis_index("c")
    pltpu.sync_copy(table_hbm, table_v)
    row0 = TC_LOOKUP_ROWS + wid * SC_ROWS_PER_W

    def src(k):
        kc = jnp.minimum(k, SC_NCHUNK - 1)
        return direct_hbm.at[pl.ds(row0 + kc * CH_ROWS, CH_ROWS), :]

    def compute(buf, a):
        for r in range(CH_ROWS):
            def vec_body(i, acc):
                d = buf[r, pl.ds(i * L, L)]
                idx = jnp.clip(d * 255.0, 0.0, 255.0).astype(jnp.int32)
                t = plsc.load_gather(table_v, [idx])
                diff = d - t
                return acc + diff * diff

            a = plsc.parallel_loop(0, VECS_PER_ROW, 1, unroll=8,
                                   carry=a)(vec_body)
        return a

    pltpu.async_copy(src(0), buf0, sem0)
    pltpu.async_copy(src(1), buf1, sem1)

    def outer(k2, acc):
        k = 2 * k2
        pltpu.make_async_copy(src(k), buf0, sem0).wait()
        acc = compute(buf0, acc)
        pltpu.async_copy(src(k + 2), buf0, sem0)
        pltpu.make_async_copy(src(k + 1), buf1, sem1).wait()
        acc = compute(buf1, acc)
        pltpu.async_copy(src(k + 3), buf1, sem1)
        return acc

    acc = lax.fori_loop(0, SC_NCHUNK // 2, outer, jnp.zeros((L,), jnp.float32))
    pltpu.make_async_copy(src(SC_NCHUNK), buf0, sem0).wait()
    pltpu.make_async_copy(src(SC_NCHUNK + 1), buf1, sem1).wait()
    out_v[...] = acc
    pltpu.sync_copy(out_v, out_hbm.at[wid])


def _tc_body(d_ref, tbl_ref, out_ref, acc_ref):
    d = d_ref[...]
    r = d * 50.0
    neg = jnp.where(d > -BETA, r * r, -1000.0 * d - 100.0)
    p = jnp.where(d >= 0.0, d, neg)
    s = jnp.sum(p)

    @pl.when(pl.program_id(0) == 0)
    def _init():
        acc_ref[0] = 0.0

    acc_ref[0] += s

    @pl.when(pl.program_id(0) < TC_LOOKUP_BLOCKS)
    def _lookup():
        tlo = tbl_ref[:, :128]
        thi = tbl_ref[:, 128:]

        def row_body(rr, s2):
            d8 = d_ref[rr]
            idx = jnp.clip(d8 * 255.0, 0.0, 255.0).astype(jnp.int32)
            low = idx & 127
            glo = jnp.take_along_axis(tlo, low, axis=1)
            ghi = jnp.take_along_axis(thi, low, axis=1)
            t = jnp.where(idx < 128, glo, ghi)
            diff = d8 - t
            return s2 + jnp.sum(diff * diff)

        acc_ref[0] += lax.fori_loop(0, TC_BM // SUB, row_body,
                                    jnp.float32(0.0))

    @pl.when(pl.program_id(0) == pl.num_programs(0) - 1)
    def _fin():
        out_ref[0] = acc_ref[0]


@jax.jit
def kernel(direct, table):
    d2 = direct.reshape(ROWS, COLS)

    sc_partials = pl.kernel(
        _sc_lookup_body,
        out_type=jax.ShapeDtypeStruct((NW, L), jnp.float32),
        mesh=plsc.VectorSubcoreMesh(
            core_axis_name="c", subcore_axis_name="s",
            num_cores=NC, num_subcores=NS),
        scratch_types=[
            pltpu.VMEM((TABLE_N,), jnp.float32),
            pltpu.VMEM((CH_ROWS, COLS), jnp.float32),
            pltpu.VMEM((CH_ROWS, COLS), jnp.float32),
            pltpu.VMEM((L,), jnp.float32),
            pltpu.SemaphoreType.DMA,
            pltpu.SemaphoreType.DMA,
        ],
        compiler_params=pltpu.CompilerParams(needs_layout_passes=False),
    )(d2, table)

    tbl8 = jnp.broadcast_to(table, (SUB, TABLE_N))
    d3 = direct.reshape(ROWS // SUB, SUB, COLS)
    tc_sum = pl.pallas_call(
        _tc_body,
        grid=(TC_GRID,),
        in_specs=[
            pl.BlockSpec((TC_BM // SUB, SUB, COLS), lambda i: (i, 0, 0)),
            pl.BlockSpec((SUB, TABLE_N), lambda i: (0, 0)),
        ],
        out_specs=pl.BlockSpec(memory_space=pltpu.SMEM),
        out_shape=jax.ShapeDtypeStruct((1,), jnp.float32),
        scratch_shapes=[pltpu.SMEM((1,), jnp.float32)],
    )(d3, tbl8)

    return (jnp.sum(sc_partials) + tc_sum[0]) / N_TOTAL


# speedup vs baseline: 1.0837x; 1.0009x over previous
import functools

import jax
import jax.numpy as jnp
from jax import lax
from jax.experimental import pallas as pl
from jax.experimental.pallas import tpu as pltpu
from jax.experimental.pallas import tpu_sc as plsc

TABLE_N = 256
BETA = 0.2

NC, NS, L = 2, 16, 16
NW = NC * NS

ROWS = 16384
COLS = 2048
N_TOTAL = ROWS * COLS

TC_LOOKUP_ROWS = 2560
SC_ROWS = ROWS - TC_LOOKUP_ROWS
SC_ROWS_PER_W = SC_ROWS // NW
CH_ROWS = 8
SC_NCHUNK = SC_ROWS_PER_W // CH_ROWS
VECS_PER_ROW = COLS // L

TC_BM = 512
TC_GRID = ROWS // TC_BM
TC_LOOKUP_BLOCKS = TC_LOOKUP_ROWS // TC_BM
SUB = 8


def _sc_lookup_body(direct_hbm, table_hbm, out_hbm, table_v, buf0, buf1,
                    out_v, sem0, sem1):
    wid = lax.axis_index("s") * NC + lax.axis_index("c")
    pltpu.sync_copy(table_hbm, table_v)
    row0 = TC_LOOKUP_ROWS + wid * SC_ROWS_PER_W

    def src(k):
        kc = jnp.minimum(k, SC_NCHUNK - 1)
        return direct_hbm.at[pl.ds(row0 + kc * CH_ROWS, CH_ROWS), :]

    def compute(buf, a):
        for r in range(CH_ROWS):
            def vec_body(i, acc):
                d = buf[r, pl.ds(i * L, L)]
                idx = jnp.clip(d * 255.0, 0.0, 255.0).astype(jnp.int32)
                t = plsc.load_gather(table_v, [idx])
                diff = d - t
                return acc + diff * diff

            a = plsc.parallel_loop(0, VECS_PER_ROW, 1, unroll=8,
                                   carry=a)(vec_body)
        return a

    pltpu.async_copy(src(0), buf0, sem0)
    pltpu.async_copy(src(1), buf1, sem1)

    def outer(k2, acc):
        k = 2 * k2
        pltpu.make_async_copy(src(k), buf0, sem0).wait()
        acc = compute(buf0, acc)
        pltpu.async_copy(src(k + 2), buf0, sem0)
        pltpu.make_async_copy(src(k + 1), buf1, sem1).wait()
        acc = compute(buf1, acc)
        pltpu.async_copy(src(k + 3), buf1, sem1)
        return acc

    acc = lax.fori_loop(0, SC_NCHUNK // 2, outer, jnp.zeros((L,), jnp.float32))
    pltpu.make_async_copy(src(SC_NCHUNK), buf0, sem0).wait()
    pltpu.make_async_copy(src(SC_NCHUNK + 1), buf1, sem1).wait()
    out_v[...] = acc
    pltpu.sync_copy(out_v, out_hbm.at[wid])


def _tc_body(d_ref, tbl_ref, out_ref, acc_ref):
    d = d_ref[...]
    r = d * 50.0
    neg = jnp.where(d > -BETA, r * r, -1000.0 * d - 100.0)
    p = jnp.where(d >= 0.0, d, neg)
    s = jnp.sum(p)

    @pl.when(pl.program_id(0) == 0)
    def _init():
        acc_ref[0] = 0.0

    acc_ref[0] += s

    @pl.when(pl.program_id(0) < TC_LOOKUP_BLOCKS)
    def _lookup():
        tlo = tbl_ref[:, :128]
        thi = tbl_ref[:, 128:]

        def row_body(rr, s2):
            d8 = d_ref[rr]
            idx = jnp.clip(d8 * 255.0, 0.0, 255.0).astype(jnp.int32)
            low = idx & 127
            glo = jnp.take_along_axis(tlo, low, axis=1)
            ghi = jnp.take_along_axis(thi, low, axis=1)
            t = jnp.where(idx < 128, glo, ghi)
            diff = d8 - t
            return s2 + jnp.sum(diff * diff)

        acc_ref[0] += lax.fori_loop(0, TC_BM // SUB, row_body,
                                    jnp.float32(0.0))

    @pl.when(pl.program_id(0) == pl.num_programs(0) - 1)
    def _fin():
        out_ref[0] = acc_ref[0]


@jax.jit
def kernel(direct, table):
    d2 = direct.reshape(ROWS, COLS)

    sc_partials = pl.kernel(
        _sc_lookup_body,
        out_type=jax.ShapeDtypeStruct((NW, L), jnp.float32),
        mesh=plsc.VectorSubcoreMesh(
            core_axis_name="c", subcore_axis_name="s",
            num_cores=NC, num_subcores=NS),
        scratch_types=[
            pltpu.VMEM((TABLE_N,), jnp.float32),
            pltpu.VMEM((CH_ROWS, COLS), jnp.float32),
            pltpu.VMEM((CH_ROWS, COLS), jnp.float32),
            pltpu.VMEM((L,), jnp.float32),
            pltpu.SemaphoreType.DMA,
            pltpu.SemaphoreType.DMA,
        ],
        compiler_params=pltpu.CompilerParams(
            needs_layout_passes=False, disable_semaphore_checks=True),
    )(d2, table)

    tbl8 = jnp.broadcast_to(table, (SUB, TABLE_N))
    d3 = direct.reshape(ROWS // SUB, SUB, COLS)
    tc_sum = pl.pallas_call(
        _tc_body,
        grid=(TC_GRID,),
        in_specs=[
            pl.BlockSpec((TC_BM // SUB, SUB, COLS), lambda i: (i, 0, 0)),
            pl.BlockSpec((SUB, TABLE_N), lambda i: (0, 0)),
        ],
        out_specs=pl.BlockSpec(memory_space=pltpu.SMEM),
        out_shape=jax.ShapeDtypeStruct((1,), jnp.float32),
        scratch_shapes=[pltpu.SMEM((1,), jnp.float32)],
    )(d3, tbl8)

    return (jnp.sum(sc_partials) + tc_sum[0]) / N_TOTAL
